# initial kernel scaffold (unmeasured)
import jax
import jax.numpy as jnp
from jax import lax
from jax.experimental import pallas as pl
from jax.experimental.pallas import tpu as pltpu

S = 2048
K = 4096
N = 8192
HALF = 1024
NT = 8
TM = HALF // NT


def _matmul(A, Wo):
    bm, bn = HALF, 512

    def body(a_ref, w_ref, o_ref):
        o_ref[...] = jnp.dot(a_ref[...], w_ref[...],
                             preferred_element_type=jnp.float32)

    return pl.pallas_call(
        body,
        grid=(S // bm, N // bn),
        in_specs=[
            pl.BlockSpec((bm, K), lambda i, j: (i, 0)),
            pl.BlockSpec((K, bn), lambda i, j: (0, j)),
        ],
        out_specs=pl.BlockSpec((bm, bn), lambda i, j: (i, j)),
        out_shape=jax.ShapeDtypeStruct((S, N), jnp.float32),
    )(A, Wo)


def _exchange_add(P):
    def body(p_any_ref, p_own_ref, o_ref, recv_ref, send_sem, recv_sem):
        t = pl.program_id(0)
        my_x = lax.axis_index("x")
        my_y = lax.axis_index("y")
        peer = (my_x, 1 - my_y)

        rdma = pltpu.make_async_remote_copy(
            src_ref=p_any_ref.at[pl.ds(HALF, HALF), :],
            dst_ref=recv_ref,
            send_sem=send_sem,
            recv_sem=recv_sem,
            device_id=peer,
            device_id_type=pl.DeviceIdType.MESH,
        )

        @pl.when(t == 0)
        def _():
            barrier_sem = pltpu.get_barrier_semaphore()
            pl.semaphore_signal(barrier_sem, inc=1, device_id=peer,
                                device_id_type=pl.DeviceIdType.MESH)
            pl.semaphore_wait(barrier_sem, 1)
            rdma.start()
            rdma.wait()

        o_ref[...] = p_own_ref[...] + recv_ref[pl.ds(t * TM, TM), :]

    return pl.pallas_call(
        body,
        grid=(NT,),
        in_specs=[
            pl.BlockSpec(memory_space=pltpu.ANY),
            pl.BlockSpec((TM, N), lambda t: (t, 0)),
        ],
        out_specs=pl.BlockSpec((TM, N), lambda t: (t, 0)),
        out_shape=jax.ShapeDtypeStruct((HALF, N), jnp.float32),
        scratch_shapes=[
            pltpu.VMEM((HALF, N), jnp.float32),
            pltpu.SemaphoreType.DMA,
            pltpu.SemaphoreType.DMA,
        ],
        compiler_params=pltpu.CompilerParams(
            collective_id=0, dimension_semantics=("arbitrary",)),
    )(P, P)


def kernel(O, Wo):
    my_y = lax.axis_index("y")
    A = O.reshape(S, K)
    A_own = lax.dynamic_slice_in_dim(A, my_y * HALF, HALF, axis=0)
    A_peer = lax.dynamic_slice_in_dim(A, (1 - my_y) * HALF, HALF, axis=0)
    A2 = jnp.concatenate([A_own, A_peer], axis=0)
    P = _matmul(A2, Wo)
    out = _exchange_add(P)
    return out.reshape(1, HALF, N)


# baseline (device time: 641137 ns/iter reference)
import jax
import jax.numpy as jnp
from jax import lax
from jax.experimental import pallas as pl
from jax.experimental.pallas import tpu as pltpu

S = 2048
K = 4096
N = 8192
HALF = 1024
NT = 8
TM = HALF // NT


def _matmul(A, Wo):
    bm, bn = HALF, 512

    def body(a_ref, w_ref, o_ref):
        o_ref[...] = jnp.dot(a_ref[...], w_ref[...],
                             preferred_element_type=jnp.float32)

    return pl.pallas_call(
        body,
        grid=(S // bm, N // bn),
        in_specs=[
            pl.BlockSpec((bm, K), lambda i, j: (i, 0)),
            pl.BlockSpec((K, bn), lambda i, j: (0, j)),
        ],
        out_specs=pl.BlockSpec((bm, bn), lambda i, j: (i, j)),
        out_shape=jax.ShapeDtypeStruct((S, N), jnp.float32),
        compiler_params=pltpu.CompilerParams(
            vmem_limit_bytes=100 * 1024 * 1024),
    )(A, Wo)


def _exchange_add(P):
    def body(p_any_ref, p_own_ref, o_ref, recv_ref, send_sem, recv_sem):
        t = pl.program_id(0)
        my_x = lax.axis_index("x")
        my_y = lax.axis_index("y")
        peer = (my_x, 1 - my_y)

        rdma = pltpu.make_async_remote_copy(
            src_ref=p_any_ref.at[pl.ds(HALF, HALF), :],
            dst_ref=recv_ref,
            send_sem=send_sem,
            recv_sem=recv_sem,
            device_id=peer,
            device_id_type=pl.DeviceIdType.MESH,
        )

        @pl.when(t == 0)
        def _():
            barrier_sem = pltpu.get_barrier_semaphore()
            pl.semaphore_signal(barrier_sem, inc=1, device_id=peer,
                                device_id_type=pl.DeviceIdType.MESH)
            pl.semaphore_wait(barrier_sem, 1)
            rdma.start()
            rdma.wait()

        o_ref[...] = p_own_ref[...] + recv_ref[pl.ds(t * TM, TM), :]

    return pl.pallas_call(
        body,
        grid=(NT,),
        in_specs=[
            pl.BlockSpec(memory_space=pl.ANY),
            pl.BlockSpec((TM, N), lambda t: (t, 0)),
        ],
        out_specs=pl.BlockSpec((TM, N), lambda t: (t, 0)),
        out_shape=jax.ShapeDtypeStruct((HALF, N), jnp.float32),
        scratch_shapes=[
            pltpu.VMEM((HALF, N), jnp.float32),
            pltpu.SemaphoreType.DMA,
            pltpu.SemaphoreType.DMA,
        ],
        compiler_params=pltpu.CompilerParams(
            collective_id=0, dimension_semantics=("arbitrary",),
            vmem_limit_bytes=100 * 1024 * 1024),
    )(P, P)


def kernel(O, Wo):
    my_y = lax.axis_index("y")
    A = O.reshape(S, K)
    A_own = lax.dynamic_slice_in_dim(A, my_y * HALF, HALF, axis=0)
    A_peer = lax.dynamic_slice_in_dim(A, (1 - my_y) * HALF, HALF, axis=0)
    A2 = jnp.concatenate([A_own, A_peer], axis=0)
    P = _matmul(A2, Wo)
    out = _exchange_add(P)
    return out.reshape(1, HALF, N)


# device time: 196193 ns/iter; 3.2679x vs baseline; 3.2679x over previous
import jax
import jax.numpy as jnp
from jax import lax
from jax.experimental import pallas as pl
from jax.experimental.pallas import tpu as pltpu

S = 2048
K = 4096
N = 8192
HALF = 1024
CW = 512
NC = N // CW
NH = NC // 2
STEPS = 3 * NH


def _c16(t, s):
    mx = s[0]
    mine = NH * mx
    other = NH * (1 - mx)
    return jnp.where(t < NH, mine + t,
                     jnp.where(t < 2 * NH, mine + t - NH, other + t - 2 * NH))


def _fused(scalars, A, Wo):
    def body(s_ref, a_ref, w_ref, o_ref, send_buf, recv_buf,
             y_send_sems, fwd_send_sems, recv_sems):
        t = pl.program_id(0)
        mx = s_ref[0]
        my = s_ref[1]
        y_peer = (mx, 1 - my)
        x_peer = (1 - mx, my)

        @pl.when(t == 0)
        def _():
            barrier_sem = pltpu.get_barrier_semaphore()
            for nbr in (y_peer, x_peer):
                pl.semaphore_signal(barrier_sem, inc=1, device_id=nbr,
                                    device_id_type=pl.DeviceIdType.MESH)
            pl.semaphore_wait(barrier_sem, 2)

        chunk = jnp.dot(a_ref[...], w_ref[...].astype(jnp.bfloat16),
                        preferred_element_type=jnp.float32)

        send_slot = NH * mx + t
        recv_slot = jnp.where(t < 2 * NH, NH * mx + t - NH,
                              NH * (1 - mx) + t - 2 * NH)

        @pl.when(t < NH)
        def _():
            send_buf[t] = chunk.astype(jnp.bfloat16)
            rdma = pltpu.make_async_remote_copy(
                src_ref=send_buf.at[t],
                dst_ref=recv_buf.at[send_slot],
                send_sem=y_send_sems.at[t],
                recv_sem=recv_sems.at[send_slot],
                device_id=y_peer,
                device_id_type=pl.DeviceIdType.MESH,
            )
            rdma.start()

        @pl.when(t >= NH)
        def _():
            recv_desc = pltpu.make_async_remote_copy(
                src_ref=recv_buf.at[recv_slot],
                dst_ref=recv_buf.at[recv_slot],
                send_sem=y_send_sems.at[0],
                recv_sem=recv_sems.at[recv_slot],
                device_id=y_peer,
                device_id_type=pl.DeviceIdType.MESH,
            )
            recv_desc.wait_recv()

            @pl.when(t < 2 * NH)
            def _():
                fwd = pltpu.make_async_remote_copy(
                    src_ref=recv_buf.at[recv_slot],
                    dst_ref=recv_buf.at[recv_slot],
                    send_sem=fwd_send_sems.at[t - NH],
                    recv_sem=recv_sems.at[recv_slot],
                    device_id=x_peer,
                    device_id_type=pl.DeviceIdType.MESH,
                )
                fwd.start()

            o_ref[...] = chunk + recv_buf[recv_slot].astype(jnp.float32)

        @pl.when(t == STEPS - 1)
        def _():
            for c in range(NH):
                pltpu.make_async_remote_copy(
                    src_ref=send_buf.at[c],
                    dst_ref=recv_buf.at[c],
                    send_sem=y_send_sems.at[c],
                    recv_sem=recv_sems.at[c],
                    device_id=y_peer,
                    device_id_type=pl.DeviceIdType.MESH,
                ).wait_send()
                pltpu.make_async_remote_copy(
                    src_ref=recv_buf.at[NH * mx + c],
                    dst_ref=recv_buf.at[NH * mx + c],
                    send_sem=fwd_send_sems.at[c],
                    recv_sem=recv_sems.at[c],
                    device_id=x_peer,
                    device_id_type=pl.DeviceIdType.MESH,
                ).wait_send()

    grid_spec = pltpu.PrefetchScalarGridSpec(
        num_scalar_prefetch=1,
        grid=(STEPS,),
        in_specs=[
            pl.BlockSpec(
                (HALF, K),
                lambda t, s: (jnp.where(t < NH, 1 - s[1], s[1]), 0)),
            pl.BlockSpec((K, CW), lambda t, s: (0, _c16(t, s))),
        ],
        out_specs=pl.BlockSpec(
            (HALF, CW),
            lambda t, s: (0, jnp.where(t < NH, NH * s[0], _c16(t, s)))),
        scratch_shapes=[
            pltpu.VMEM((NH, HALF, CW), jnp.bfloat16),
            pltpu.VMEM((NC, HALF, CW), jnp.bfloat16),
            pltpu.SemaphoreType.DMA((NH,)),
            pltpu.SemaphoreType.DMA((NH,)),
            pltpu.SemaphoreType.DMA((NC,)),
        ],
    )
    return pl.pallas_call(
        body,
        grid_spec=grid_spec,
        out_shape=jax.ShapeDtypeStruct((HALF, N), jnp.float32),
        compiler_params=pltpu.CompilerParams(
            collective_id=0,
            dimension_semantics=("arbitrary",),
            vmem_limit_bytes=100 * 1024 * 1024),
    )(scalars, A, Wo)


def kernel(O, Wo):
    mx = lax.axis_index("x")
    my = lax.axis_index("y")
    A = O.reshape(S, K).astype(jnp.bfloat16)
    scalars = jnp.stack([mx, my]).astype(jnp.int32)
    out = _fused(scalars, A, Wo)
    return out.reshape(1, HALF, N)
